# baseline (speedup 1.0000x reference)
import jax
import jax.numpy as jnp
from jax import lax
from jax.experimental import pallas as pl
from jax.experimental.pallas import tpu as pltpu
from jax.experimental.pallas import tpu_sc as plsc

_N = 8192
_NC = 2
_NS = 16

_SP_RPW = 352
_SP_CR = 4
_SP_NCHUNK = _SP_RPW // _SP_CR
_SP_NBUF = 3
_SP_PF = 2

_TS_R0 = 16 * _SP_RPW
_TS_RPW = 160
_TS_CR = 1
_TS_NCHUNK = _TS_RPW // _TS_CR
_TS_NBUF = 4
_TS_PF = 2


def _win(r0, c):
    base = r0 + c * _SP_CR
    beta = lax.rem(base, 128)
    return pl.multiple_of(base - beta, 128), beta


def _spmem_path(x_hbm, v_hbm, out_hbm, buf, win, vals, in_sems, out_sems,
                wi_sems, wo_sems, spid):
    r0 = spid * _SP_RPW
    slot = spid % 8

    pltpu.make_async_copy(
        v_hbm.at[pl.ds(r0, _SP_RPW)], vals.at[pl.ds(0, _SP_RPW)], in_sems.at[0]
    ).start()
    pltpu.make_async_copy(
        v_hbm.at[pl.ds(r0, _SP_RPW)], vals.at[pl.ds(0, _SP_RPW)], in_sems.at[0]
    ).wait()

    def start_in(c, b):
        pltpu.make_async_copy(
            x_hbm.at[pl.ds(r0 + c * _SP_CR, _SP_CR), :], buf.at[slot, b],
            in_sems.at[b],
        ).start()
        w, _ = _win(r0, c)
        pltpu.make_async_copy(
            x_hbm.at[pl.ds(r0 + c * _SP_CR, _SP_CR), pl.ds(w, 128)],
            win.at[b],
            wi_sems.at[b],
        ).start()

    def wait_in(b):
        pltpu.make_async_copy(
            x_hbm.at[pl.ds(r0, _SP_CR), :], buf.at[slot, b], in_sems.at[b]
        ).wait()
        pltpu.make_async_copy(
            x_hbm.at[pl.ds(r0, _SP_CR), pl.ds(0, 128)], win.at[b], wi_sems.at[b]
        ).wait()

    def start_out(c, b):
        pltpu.make_async_copy(
            buf.at[slot, b], out_hbm.at[pl.ds(r0 + c * _SP_CR, _SP_CR), :],
            out_sems.at[b],
        ).start()

    def wait_out(b):
        pltpu.make_async_copy(
            buf.at[slot, b], out_hbm.at[pl.ds(r0, _SP_CR), :], out_sems.at[b]
        ).wait()

    for b in range(_SP_PF):
        start_in(b, b)

    iota = lax.broadcasted_iota(jnp.int32, (16,), 0)
    diag_mask = iota < _SP_CR

    def chunk_body(c, b):
        wait_in(b)
        w, beta = _win(r0, c)
        vals_v = plsc.load_gather(vals, [c * _SP_CR + iota])
        plsc.addupdate_scatter(win.at[b], [iota, beta + iota], vals_v, mask=diag_mask)
        pltpu.make_async_copy(
            win.at[b], buf.at[slot, b, :, pl.ds(w, 128)], wo_sems.at[b]
        ).start()
        nb = (b + _SP_PF) % _SP_NBUF

        @pl.when(c + _SP_PF < _SP_NCHUNK)
        def _():
            @pl.when(c + _SP_PF >= _SP_NBUF)
            def _():
                wait_out(nb)

            start_in(c + _SP_PF, nb)

        pltpu.make_async_copy(
            win.at[b], buf.at[slot, b, :, pl.ds(0, 128)], wo_sems.at[b]
        ).wait()
        start_out(c, b)

    def outer(o, _):
        for b in range(_SP_NBUF):
            chunk_body(o * _SP_NBUF + b, b)
        return ()

    lax.fori_loop(0, _SP_NCHUNK // _SP_NBUF, outer, ())

    for c in range((_SP_NCHUNK // _SP_NBUF) * _SP_NBUF, _SP_NCHUNK):
        chunk_body(c, c % _SP_NBUF)

    for b in range(_SP_NBUF):
        wait_out(b)


def _tilespmem_path(x_hbm, v_hbm, out_hbm, buf, vals, in_sems, out_sems, tsid):
    r0 = _TS_R0 + tsid * _TS_RPW

    pltpu.make_async_copy(
        v_hbm.at[pl.ds(r0, _TS_RPW)], vals.at[pl.ds(0, _TS_RPW)], in_sems.at[0]
    ).start()
    pltpu.make_async_copy(
        v_hbm.at[pl.ds(r0, _TS_RPW)], vals.at[pl.ds(0, _TS_RPW)], in_sems.at[0]
    ).wait()

    def start_in(c, b):
        pltpu.make_async_copy(
            x_hbm.at[pl.ds(r0 + c * _TS_CR, _TS_CR), :], buf.at[b], in_sems.at[b]
        ).start()

    def wait_in(b):
        pltpu.make_async_copy(
            x_hbm.at[pl.ds(r0, _TS_CR), :], buf.at[b], in_sems.at[b]
        ).wait()

    def start_out(c, b):
        pltpu.make_async_copy(
            buf.at[b], out_hbm.at[pl.ds(r0 + c * _TS_CR, _TS_CR), :], out_sems.at[b]
        ).start()

    def wait_out(b):
        pltpu.make_async_copy(
            buf.at[b], out_hbm.at[pl.ds(r0, _TS_CR), :], out_sems.at[b]
        ).wait()

    for b in range(_TS_PF):
        start_in(b, b)

    iota = lax.broadcasted_iota(jnp.int32, (16,), 0)
    diag_mask = iota < _TS_CR

    def outer(o, _):
        for b in range(_TS_NBUF):
            c = o * _TS_NBUF + b
            wait_in(b)
            vals_v = plsc.load_gather(vals, [c * _TS_CR + iota])
            col0 = r0 + c * _TS_CR
            plsc.addupdate_scatter(
                buf.at[b], [iota, col0 + iota], vals_v, mask=diag_mask
            )
            start_out(c, b)
            nb = (b + _TS_PF) % _TS_NBUF

            @pl.when(c + _TS_PF < _TS_NCHUNK)
            def _():
                @pl.when(c + _TS_PF >= _TS_NBUF)
                def _():
                    wait_out(nb)

                start_in(c + _TS_PF, nb)

        return ()

    lax.fori_loop(0, _TS_NCHUNK // _TS_NBUF, outer, ())

    for b in range(_TS_NBUF):
        wait_out(b)


def _sc_body(x_hbm, v_hbm, out_hbm, buf_sp, win, vals_sp, buf_ts, vals_ts,
             sp_in, sp_out, sp_wi, sp_wo, ts_in, ts_out):
    sid = lax.axis_index("s")
    cid = lax.axis_index("c")
    is_sp = sid < 8

    @pl.when(is_sp)
    def _():
        _spmem_path(
            x_hbm, v_hbm, out_hbm, buf_sp, win, vals_sp, sp_in, sp_out,
            sp_wi, sp_wo, cid * 8 + sid,
        )

    @pl.when(jnp.logical_not(is_sp))
    def _():
        _tilespmem_path(
            x_hbm, v_hbm, out_hbm, buf_ts, vals_ts, ts_in, ts_out,
            cid * 8 + (sid - 8),
        )


def kernel(x, values):
    mesh = plsc.VectorSubcoreMesh(
        core_axis_name="c", subcore_axis_name="s", num_cores=_NC, num_subcores=_NS
    )
    f = pl.kernel(
        _sc_body,
        out_type=jax.ShapeDtypeStruct((_N, _N), jnp.float32),
        mesh=mesh,
        scratch_types=[
            pltpu.MemorySpace.VMEM_SHARED((8, _SP_NBUF, _SP_CR, _N), jnp.float32),
            pltpu.VMEM((_SP_NBUF, _SP_CR, 128), jnp.float32),
            pltpu.VMEM((_SP_RPW + 16,), jnp.float32),
            pltpu.VMEM((_TS_NBUF, _TS_CR, _N), jnp.float32),
            pltpu.VMEM((_TS_RPW + 16,), jnp.float32),
            pltpu.SemaphoreType.DMA((_SP_NBUF,)),
            pltpu.SemaphoreType.DMA((_SP_NBUF,)),
            pltpu.SemaphoreType.DMA((_SP_NBUF,)),
            pltpu.SemaphoreType.DMA((_SP_NBUF,)),
            pltpu.SemaphoreType.DMA((_TS_NBUF,)),
            pltpu.SemaphoreType.DMA((_TS_NBUF,)),
        ],
        compiler_params=pltpu.CompilerParams(needs_layout_passes=False),
    )
    return f(x, values)


# speedup vs baseline: 1.0565x; 1.0565x over previous
import jax
import jax.numpy as jnp
from jax import lax
from jax.experimental import pallas as pl
from jax.experimental.pallas import tpu as pltpu
from jax.experimental.pallas import tpu_sc as plsc

_N = 8192
_NC = 2
_NS = 16
_NW = _NC * _NS
_RPW = _N // _NW
_CR = 2
_NCHUNK = _RPW // _CR
_NBUF = 4
_PF = 2


def _win_start(r0, c):
    base = r0 + c * _CR
    beta = lax.rem(base, 128)
    return pl.multiple_of(base - beta, 128), beta


def _sc_body(x_hbm, v_hbm, out_hbm, buf, win, vals, in_sems, out_sems, wi_sems, wo_sems):
    sid = lax.axis_index("s")
    wid = lax.axis_index("c") * _NS + sid
    r0 = wid * _RPW

    pltpu.make_async_copy(
        v_hbm.at[pl.ds(r0, _RPW)], vals.at[pl.ds(0, _RPW)], in_sems.at[0]
    ).start()
    pltpu.make_async_copy(
        v_hbm.at[pl.ds(r0, _RPW)], vals.at[pl.ds(0, _RPW)], in_sems.at[0]
    ).wait()

    def start_in(c, b):
        pltpu.make_async_copy(
            x_hbm.at[pl.ds(r0 + c * _CR, _CR), :], buf.at[sid, b], in_sems.at[b]
        ).start()
        w, _ = _win_start(r0, c)
        pltpu.make_async_copy(
            x_hbm.at[pl.ds(r0 + c * _CR, _CR), pl.ds(w, 128)],
            win.at[b],
            wi_sems.at[b],
        ).start()

    def wait_in(b):
        pltpu.make_async_copy(
            x_hbm.at[pl.ds(r0, _CR), :], buf.at[sid, b], in_sems.at[b]
        ).wait()
        pltpu.make_async_copy(
            x_hbm.at[pl.ds(r0, _CR), pl.ds(0, 128)], win.at[b], wi_sems.at[b]
        ).wait()

    def start_out(c, b):
        pltpu.make_async_copy(
            buf.at[sid, b], out_hbm.at[pl.ds(r0 + c * _CR, _CR), :], out_sems.at[b]
        ).start()

    def wait_out(b):
        pltpu.make_async_copy(
            buf.at[sid, b], out_hbm.at[pl.ds(r0, _CR), :], out_sems.at[b]
        ).wait()

    for b in range(_PF):
        start_in(b, b)

    iota = lax.broadcasted_iota(jnp.int32, (16,), 0)
    diag_mask = iota < _CR

    def outer(o, _):
        for b in range(_NBUF):
            c = o * _NBUF + b
            wait_in(b)
            w, beta = _win_start(r0, c)
            vals_v = plsc.load_gather(vals, [c * _CR + iota])
            plsc.addupdate_scatter(
                win.at[b], [iota, beta + iota], vals_v, mask=diag_mask
            )
            pltpu.make_async_copy(
                win.at[b], buf.at[sid, b, :, pl.ds(w, 128)], wo_sems.at[b]
            ).start()
            pltpu.make_async_copy(
                win.at[b], buf.at[sid, b, :, pl.ds(0, 128)], wo_sems.at[b]
            ).wait()
            start_out(c, b)
            nb = (b + _PF) % _NBUF

            @pl.when(c + _PF < _NCHUNK)
            def _():
                @pl.when(c + _PF >= _NBUF)
                def _():
                    wait_out(nb)

                start_in(c + _PF, nb)

        return ()

    lax.fori_loop(0, _NCHUNK // _NBUF, outer, ())

    for b in range(_NBUF):
        wait_out(b)


def kernel(x, values):
    mesh = plsc.VectorSubcoreMesh(
        core_axis_name="c", subcore_axis_name="s", num_cores=_NC, num_subcores=_NS
    )
    f = pl.kernel(
        _sc_body,
        out_type=jax.ShapeDtypeStruct((_N, _N), jnp.float32),
        mesh=mesh,
        scratch_types=[
            pltpu.MemorySpace.VMEM_SHARED((_NS, _NBUF, _CR, _N), jnp.float32),
            pltpu.VMEM((_NBUF, _CR, 128), jnp.float32),
            pltpu.VMEM((_RPW + 16,), jnp.float32),
            pltpu.SemaphoreType.DMA((_NBUF,)),
            pltpu.SemaphoreType.DMA((_NBUF,)),
            pltpu.SemaphoreType.DMA((_NBUF,)),
            pltpu.SemaphoreType.DMA((_NBUF,)),
        ],
        compiler_params=pltpu.CompilerParams(needs_layout_passes=False),
    )
    return f(x, values)
